# Initial kernel scaffold; baseline (speedup 1.0000x reference)
import jax
import jax.numpy as jnp
from jax.experimental import pallas as pl


def kernel(input_ids, token_type_ids, word_table, pos_table, type_table, gamma, beta):
    raise NotImplementedError("write your pallas kernel here")


# speedup vs baseline: 1.2434x; 1.2434x over previous
import functools

import jax
import jax.numpy as jnp
from jax import lax
from jax.experimental import pallas as pl
from jax.experimental.pallas import tpu as pltpu
from jax.experimental.pallas import tpu_sc as plsc

VOCAB = 100000
HIDDEN = 128
MAX_POS = 512
EPS = 1e-12

NC, NS, L = 2, 16, 16
NW = NC * NS
N_TOK = 64 * 512
TPW = N_TOK // NW
C = 128
NCHUNK = TPW // C
NSL = HIDDEN // L
W = 17


def _tec_body(ids_hbm, tt_hbm, word_hbm, pos_hbm, type_hbm, gamma_hbm,
              beta_hbm, out_hbm, pos_v, rows_v, idx_v, tt_v, type_v, g_v,
              b_v, sbuf, qbuf, sem):
    wid = lax.axis_index("s") * NC + lax.axis_index("c")
    base = wid * TPW

    pltpu.sync_copy(pos_hbm, pos_v)
    pltpu.sync_copy(type_hbm, type_v)
    pltpu.sync_copy(gamma_hbm, g_v)
    pltpu.sync_copy(beta_hbm, b_v)

    g = [g_v[pl.ds(L * j, L)] for j in range(NSL)]
    b = [b_v[pl.ds(L * j, L)] for j in range(NSL)]
    t0 = [type_v[0, pl.ds(L * j, L)] for j in range(NSL)]
    td = [type_v[1, pl.ds(L * j, L)] - type_v[0, pl.ds(L * j, L)]
          for j in range(NSL)]
    ci = lax.iota(jnp.int32, L)
    ciw = ci * W

    def chunk_body(c, carry):
        start = base + c * C
        pltpu.sync_copy(ids_hbm.at[pl.ds(start, C)], idx_v)
        pltpu.sync_copy(tt_hbm.at[pl.ds(start, C)], tt_v)
        pltpu.async_copy(word_hbm.at[idx_v], rows_v, sem).wait()
        prow_base = lax.rem(c, MAX_POS // C) * C

        def grp_body(gi, carry2):
            gbase = gi * L
            ttf = (tt_v[pl.ds(gbase, L)]).astype(jnp.float32)
            for k in range(L):
                i = gbase + k
                tf = ttf[k]
                p = prow_base + i
                s = None
                q = None
                for j in range(NSL):
                    sl = pl.ds(L * j, L)
                    x = rows_v[i, sl] + pos_v[p, sl] + t0[j] + tf * td[j]
                    rows_v[i, sl] = x
                    s = x if s is None else s + x
                    q = x * x if q is None else q + x * x
                plsc.store_scatter(sbuf, [ciw + k], s)
                plsc.store_scatter(qbuf, [ciw + k], q)
            tot = None
            totq = None
            for l in range(L):
                rl = ci + (W * l)
                vs = plsc.load_gather(sbuf, [rl])
                vq = plsc.load_gather(qbuf, [rl])
                tot = vs if tot is None else tot + vs
                totq = vq if totq is None else totq + vq
            mu = tot * (1.0 / HIDDEN)
            var = totq * (1.0 / HIDDEN) - mu * mu
            v = var + EPS
            iy = jnp.int32(0x5F3759DF) - lax.shift_right_arithmetic(
                plsc.bitcast(v, jnp.int32), 1)
            y = plsc.bitcast(iy, jnp.float32)
            h = 0.5 * v
            y = y * (1.5 - h * y * y)
            y = y * (1.5 - h * y * y)
            y = y * (1.5 - h * y * y)
            nbv = -mu * y
            for k in range(L):
                i = gbase + k
                yk = y[k]
                nk = nbv[k]
                for j in range(NSL):
                    sl = pl.ds(L * j, L)
                    x = rows_v[i, sl]
                    rows_v[i, sl] = (x * yk + nk) * g[j] + b[j]
            return carry2

        lax.fori_loop(0, C // L, grp_body, 0)
        pltpu.sync_copy(rows_v, out_hbm.at[pl.ds(start, C)])
        return carry

    lax.fori_loop(0, NCHUNK, chunk_body, 0)


@jax.jit
def _bert_embed_sc(ids_flat, tt_flat, word_table, pos_table, type_table,
                   gamma, beta):
    mesh = plsc.VectorSubcoreMesh(core_axis_name="c", subcore_axis_name="s")
    run = functools.partial(
        pl.kernel,
        out_type=jax.ShapeDtypeStruct((N_TOK, HIDDEN), jnp.float32),
        mesh=mesh,
        compiler_params=pltpu.CompilerParams(needs_layout_passes=False),
        scratch_types=[
            pltpu.VMEM((MAX_POS, HIDDEN), jnp.float32),
            pltpu.VMEM((C, HIDDEN), jnp.float32),
            pltpu.VMEM((C,), jnp.int32),
            pltpu.VMEM((C,), jnp.int32),
            pltpu.VMEM((2, HIDDEN), jnp.float32),
            pltpu.VMEM((HIDDEN,), jnp.float32),
            pltpu.VMEM((HIDDEN,), jnp.float32),
            pltpu.VMEM((L * W,), jnp.float32),
            pltpu.VMEM((L * W,), jnp.float32),
            pltpu.SemaphoreType.DMA,
        ],
    )(_tec_body)
    return run(ids_flat, tt_flat, word_table, pos_table, type_table,
               gamma, beta)


def kernel(input_ids, token_type_ids, word_table, pos_table, type_table,
           gamma, beta):
    B, S = input_ids.shape
    out = _bert_embed_sc(
        input_ids.reshape(-1).astype(jnp.int32),
        token_type_ids.reshape(-1).astype(jnp.int32),
        word_table, pos_table, type_table, gamma, beta)
    return out.reshape(B, S, HIDDEN)
